# baseline (speedup 1.0000x reference)
import functools

import numpy as np
import jax
import jax.numpy as jnp
from jax import lax
from jax.experimental import pallas as pl
from jax.experimental.pallas import tpu as pltpu
from jax.experimental.pallas import tpu_sc as plsc

N = 10000
E = 320000
D = 128
H = 16
LATENT = 7
PARAMS = 35

NC = 2
NS = 16
NW = NC * NS
SB = 128
EP = 327680
EW = EP // NW
NB = EW // SB
NPAD = 10240
RPT = NPAD // NS
NBUF = 8
NOUT = NB // NBUF

PK = 8
PROWS = NPAD // PK
OROWS = N // PK
PBLK = 128
GRID = PROWS // PBLK


def _sc_body(hw, srcm, dstm, wm, out, acc, src_v, dst_v, w_v, rows_v, msg_v,
             zb_v, gsem, ssem):
    cid = lax.axis_index("c")
    sid = lax.axis_index("s")
    wid = sid * NC + cid

    def _z(i, c):
        zb_v[i, :] = jnp.zeros((H,), jnp.float32)
        return c

    lax.fori_loop(0, RPT, _z, 0)
    pltpu.sync_copy(zb_v, acc.at[pl.ds(sid * RPT, RPT)])
    plsc.subcore_barrier()

    pltpu.sync_copy(srcm.at[wid], src_v)
    pltpu.sync_copy(dstm.at[wid], dst_v)
    pltpu.sync_copy(wm.at[wid], w_v)

    def _mul(j, b):
        for g in range(SB // H):
            w16 = w_v[j, pl.ds(g * H, H)]
            for e in range(H):
                msg_v[b, g * H + e, :] = rows_v[b, g * H + e, :] * w16[e]

    def _gather(j, b):
        pltpu.async_copy(hw.at[src_v.at[j]], rows_v.at[b], gsem.at[b])

    def _gwait(b):
        pltpu.make_async_copy(hw.at[src_v.at[0]], rows_v.at[b],
                              gsem.at[b]).wait()

    def _scatter(j, b):
        pltpu.async_copy(msg_v.at[b], acc.at[dst_v.at[j]], ssem.at[b],
                         add=True)

    def _swait(j, b):
        pltpu.make_async_copy(msg_v.at[b], acc.at[dst_v.at[j]],
                              ssem.at[b]).wait()

    for b in range(NBUF):
        _gather(b, b)
    for b in range(NBUF):
        _gwait(b)
        _mul(b, b)
        _scatter(b, b)
        _gather(b + NBUF, b)

    def _outer(o, c):
        for b in range(NBUF):
            j = o * NBUF + b
            _gwait(b)
            _swait(j, b)
            _mul(j, b)
            _scatter(j, b)
            _gather(j + NBUF, b)
        return c

    lax.fori_loop(1, NOUT - 1, _outer, 0)

    for b in range(NBUF):
        j = (NOUT - 1) * NBUF + b
        _gwait(b)
        _swait(j, b)
        _mul(j, b)
        _scatter(j, b)
    for b in range(NBUF):
        _swait(0, b)
    plsc.subcore_barrier()

    pltpu.sync_copy(acc.at[pl.ds(sid * RPT, RPT)],
                    out.at[cid, pl.ds(sid * RPT, RPT)])


@functools.lru_cache(maxsize=None)
def _get_sc_aggregate():
  return pl.kernel(
    _sc_body,
    out_type=jax.ShapeDtypeStruct((NC, NPAD, H), jnp.float32),
    mesh=plsc.VectorSubcoreMesh(core_axis_name="c", subcore_axis_name="s",
                                num_cores=NC, num_subcores=NS),
    scratch_types=[
        pltpu.VMEM_SHARED((NPAD, H), jnp.float32),
        pltpu.VMEM((NB, SB), jnp.int32),
        pltpu.VMEM((NB, SB), jnp.int32),
        pltpu.VMEM((NB, SB), jnp.float32),
        pltpu.VMEM((NBUF, SB, H), jnp.float32),
        pltpu.VMEM((NBUF, SB, H), jnp.float32),
        pltpu.VMEM((RPT, H), jnp.float32),
        pltpu.SemaphoreType.DMA((NBUF,)),
        pltpu.SemaphoreType.DMA((NBUF,)),
    ],
    compiler_params=pltpu.CompilerParams(use_tc_tiling_on_sc=False),
  )


def _mm_body(x_ref, w_ref, o_ref):
    acc = jnp.dot(x_ref[:, 0, :], w_ref[0],
                  preferred_element_type=jnp.float32)
    for k in range(1, PK):
        acc += jnp.dot(x_ref[:, k, :], w_ref[k],
                       preferred_element_type=jnp.float32)
    o_ref[...] = acc


_mm_xw1 = pl.pallas_call(
    _mm_body,
    grid=(GRID,),
    in_specs=[pl.BlockSpec((PBLK, PK, D), lambda i: (i, 0, 0)),
              pl.BlockSpec((PK, D, PK * H), lambda i: (0, 0, 0))],
    out_specs=pl.BlockSpec((PBLK, PK * H), lambda i: (i, 0)),
    out_shape=jax.ShapeDtypeStruct((PROWS, PK * H), jnp.float32),
)

_SEL = np.zeros((PK, H, PK * H), np.float32)
for _k in range(PK):
    for _f in range(H):
        _SEL[_k, _f, _k * H + _f] = 1.0


def _mid_body(p_ref, b1_ref, w2_ref, o_ref):
    h = p_ref[0] + p_ref[1] + b1_ref[...]
    h = jnp.maximum(h, 0.0)
    o_ref[...] = jnp.dot(h, w2_ref[...], preferred_element_type=jnp.float32)


_mid = pl.pallas_call(
    _mid_body,
    grid=(GRID,),
    in_specs=[pl.BlockSpec((2, PBLK, PK * H), lambda i: (0, i, 0)),
              pl.BlockSpec((1, PK * H), lambda i: (0, 0)),
              pl.BlockSpec((PK * H, PK * H), lambda i: (0, 0))],
    out_specs=pl.BlockSpec((PBLK, PK * H), lambda i: (i, 0)),
    out_shape=jax.ShapeDtypeStruct((PROWS, PK * H), jnp.float32),
)


def _head_body(p_ref, b2_ref, wd_ref, bd_ref, eps_ref, cmat_ref, smat_ref,
               dmask_ref, ones_ref, o_ref):
    h2 = p_ref[0] + p_ref[1] + b2_ref[...]
    params = jnp.dot(h2, wd_ref[...],
                     preferred_element_type=jnp.float32) + bd_ref[...]
    sp = jax.nn.softplus(params) + 1e-5
    dmask = dmask_ref[...]
    pd = dmask * sp + (1.0 - dmask) * params
    g = jnp.dot(eps_ref[...], cmat_ref[...],
                preferred_element_type=jnp.float32,
                precision=jax.lax.Precision.HIGHEST) + ones_ref[...]
    o_ref[...] = jnp.dot(pd * g, smat_ref[...],
                         preferred_element_type=jnp.float32,
                         precision=jax.lax.Precision.HIGHEST)


_head = pl.pallas_call(
    _head_body,
    grid=(GRID,),
    in_specs=[pl.BlockSpec((2, PBLK, PK * H), lambda i: (0, i, 0)),
              pl.BlockSpec((1, PK * H), lambda i: (0, 0)),
              pl.BlockSpec((PK * H, PK * PARAMS), lambda i: (0, 0)),
              pl.BlockSpec((1, PK * PARAMS), lambda i: (0, 0)),
              pl.BlockSpec((PBLK, PK * LATENT), lambda i: (i, 0)),
              pl.BlockSpec((PK * LATENT, PK * PARAMS), lambda i: (0, 0)),
              pl.BlockSpec((PK * PARAMS, PK * LATENT), lambda i: (0, 0)),
              pl.BlockSpec((1, PK * PARAMS), lambda i: (0, 0)),
              pl.BlockSpec((1, PK * PARAMS), lambda i: (0, 0))],
    out_specs=pl.BlockSpec((PBLK, PK * LATENT), lambda i: (i, 0)),
    out_shape=jax.ShapeDtypeStruct((OROWS, PK * LATENT), jnp.float32),
)

_RI, _CI = np.tril_indices(LATENT)
_CMAT = np.zeros((LATENT, PARAMS), np.float32)
_SMAT = np.zeros((PARAMS, LATENT), np.float32)
_DMASK = np.zeros((1, PARAMS), np.float32)
_ONES = np.zeros((1, PARAMS), np.float32)
_ONES[0, :LATENT] = 1.0
for _k in range(len(_RI)):
    _CMAT[_CI[_k], LATENT + _k] = 1.0
    _SMAT[LATENT + _k, _RI[_k]] = 1.0
    if _RI[_k] == _CI[_k]:
        _DMASK[0, LATENT + _k] = 1.0
for _i in range(LATENT):
    _SMAT[_i, _i] = 1.0

_EYE = np.eye(PK, dtype=np.float32)
_CMAT_P = np.kron(_EYE, _CMAT)
_SMAT_P = np.kron(_EYE, _SMAT)
_DMASK_P = np.tile(_DMASK, (1, PK))
_ONES_P = np.tile(_ONES, (1, PK))


def kernel(x, edge_index, edge_weight, eps, W1, b1, W2, b2, Wd, bd):
    pad = jnp.broadcast_to(jnp.arange(EP - E, dtype=jnp.int32) % N, (2, EP - E))
    ei = jnp.concatenate([edge_index.astype(jnp.int32), pad], axis=1)
    ewp = jnp.concatenate([edge_weight,
                           jnp.zeros((EP - E,), jnp.float32)])
    src = ei[0].reshape(NW, NB, SB)
    dst = ei[1].reshape(NW, NB, SB)
    ew = ewp.reshape(NW, NB, SB)

    eye = jnp.eye(PK, dtype=jnp.float32)
    w2p = jnp.kron(eye, W2)
    wdp = jnp.kron(eye, Wd)
    b1p = jnp.tile(b1, PK).reshape(1, PK * H)
    b2p = jnp.tile(b2, PK).reshape(1, PK * H)
    bdp = jnp.tile(bd, PK).reshape(1, PK * PARAMS)
    epsp = eps.reshape(OROWS, PK * LATENT)

    x3 = x.reshape(OROWS, PK, D)
    w1p = jnp.einsum('df,kfc->kdc', W1, jnp.asarray(_SEL))

    sc_aggregate = _get_sc_aggregate()
    hw1 = _mm_xw1(x3, w1p).reshape(NPAD, H)
    p1 = sc_aggregate(hw1, src, dst, ew).reshape(NC, PROWS, PK * H)
    hw2 = _mid(p1, b1p, w2p).reshape(NPAD, H)
    p2 = sc_aggregate(hw2, src, dst, ew).reshape(NC, PROWS, PK * H)
    out = _head(p2, b2p, wdp, bdp, epsp,
                jnp.asarray(_CMAT_P), jnp.asarray(_SMAT_P),
                jnp.asarray(_DMASK_P), jnp.asarray(_ONES_P))
    return out.reshape(N, LATENT)


# speedup vs baseline: 1.1733x; 1.1733x over previous
import functools

import numpy as np
import jax
import jax.numpy as jnp
from jax import lax
from jax.experimental import pallas as pl
from jax.experimental.pallas import tpu as pltpu
from jax.experimental.pallas import tpu_sc as plsc

N = 10000
E = 320000
D = 128
H = 16
LATENT = 7
PARAMS = 35

NC = 2
NS = 16
NW = NC * NS
SB = 128
EP = 327680
EW = EP // NW
NB = EW // SB
NPAD = 10240
RPT = NPAD // NS
NBUF = 5
NOUT = NB // NBUF

PK = 8
PROWS = NPAD // PK
OROWS = N // PK
PBLK = 128
GRID = PROWS // PBLK


def _sc_body(hw, srcm, dstm, wm, out, acc, src_v, dst_v, w_v, rows_v, msg_v,
             zb_v, gsem, ssem):
    cid = lax.axis_index("c")
    sid = lax.axis_index("s")
    wid = sid * NC + cid

    def _z(i, c):
        zb_v[i, :] = jnp.zeros((H,), jnp.float32)
        return c

    lax.fori_loop(0, RPT, _z, 0)
    pltpu.sync_copy(zb_v, acc.at[pl.ds(sid * RPT, RPT)])
    plsc.subcore_barrier()

    pltpu.sync_copy(srcm.at[wid], src_v)
    pltpu.sync_copy(dstm.at[wid], dst_v)
    pltpu.sync_copy(wm.at[wid], w_v)

    def _mul(j, b):
        for g in range(SB // H):
            w16 = w_v[j, pl.ds(g * H, H)]
            for e in range(H):
                msg_v[b, g * H + e, :] = rows_v[b, g * H + e, :] * w16[e]

    def _gather(j, b):
        pltpu.async_copy(hw.at[src_v.at[j]], rows_v.at[b], gsem.at[b])

    def _gwait(b):
        pltpu.make_async_copy(hw.at[src_v.at[0]], rows_v.at[b],
                              gsem.at[b]).wait()

    def _scatter(j, b):
        pltpu.async_copy(msg_v.at[b], acc.at[dst_v.at[j]], ssem.at[b],
                         add=True)

    def _swait(j, b):
        pltpu.make_async_copy(msg_v.at[b], acc.at[dst_v.at[j]],
                              ssem.at[b]).wait()

    for b in range(NBUF):
        _gather(b, b)
    for b in range(NBUF):
        _gwait(b)
        _mul(b, b)
        _scatter(b, b)
        _gather(b + NBUF, b)

    def _outer(o, c):
        for b in range(NBUF):
            j = o * NBUF + b
            _gwait(b)
            _swait(j, b)
            _mul(j, b)
            _scatter(j, b)
            _gather(j + NBUF, b)
        return c

    lax.fori_loop(1, NOUT - 1, _outer, 0)

    for b in range(NBUF):
        j = (NOUT - 1) * NBUF + b
        _gwait(b)
        _swait(j, b)
        _mul(j, b)
        _scatter(j, b)
    for b in range(NBUF):
        _swait(0, b)
    plsc.subcore_barrier()

    pltpu.sync_copy(acc.at[pl.ds(sid * RPT, RPT)],
                    out.at[cid, pl.ds(sid * RPT, RPT)])


@functools.lru_cache(maxsize=None)
def _get_sc_aggregate():
  return pl.kernel(
    _sc_body,
    out_type=jax.ShapeDtypeStruct((NC, NPAD, H), jnp.float32),
    mesh=plsc.VectorSubcoreMesh(core_axis_name="c", subcore_axis_name="s",
                                num_cores=NC, num_subcores=NS),
    scratch_types=[
        pltpu.VMEM_SHARED((NPAD, H), jnp.float32),
        pltpu.VMEM((NB, SB), jnp.int32),
        pltpu.VMEM((NB, SB), jnp.int32),
        pltpu.VMEM((NB, SB), jnp.float32),
        pltpu.VMEM((NBUF, SB, H), jnp.float32),
        pltpu.VMEM((NBUF, SB, H), jnp.float32),
        pltpu.VMEM((RPT, H), jnp.float32),
        pltpu.SemaphoreType.DMA((NBUF,)),
        pltpu.SemaphoreType.DMA((NBUF,)),
    ],
    compiler_params=pltpu.CompilerParams(use_tc_tiling_on_sc=False),
  )


def _mm_body(x_ref, w_ref, o_ref):
    acc = jnp.dot(x_ref[:, 0, :], w_ref[0],
                  preferred_element_type=jnp.float32)
    for k in range(1, PK):
        acc += jnp.dot(x_ref[:, k, :], w_ref[k],
                       preferred_element_type=jnp.float32)
    o_ref[...] = acc


_mm_xw1 = pl.pallas_call(
    _mm_body,
    grid=(GRID,),
    in_specs=[pl.BlockSpec((PBLK, PK, D), lambda i: (i, 0, 0)),
              pl.BlockSpec((PK, D, PK * H), lambda i: (0, 0, 0))],
    out_specs=pl.BlockSpec((PBLK, PK * H), lambda i: (i, 0)),
    out_shape=jax.ShapeDtypeStruct((PROWS, PK * H), jnp.float32),
)

_SEL = np.zeros((PK, H, PK * H), np.float32)
for _k in range(PK):
    for _f in range(H):
        _SEL[_k, _f, _k * H + _f] = 1.0


def _mid_body(p_ref, b1_ref, w2_ref, o_ref):
    h = p_ref[0] + p_ref[1] + b1_ref[...]
    h = jnp.maximum(h, 0.0)
    o_ref[...] = jnp.dot(h, w2_ref[...], preferred_element_type=jnp.float32)


_mid = pl.pallas_call(
    _mid_body,
    grid=(GRID,),
    in_specs=[pl.BlockSpec((2, PBLK, PK * H), lambda i: (0, i, 0)),
              pl.BlockSpec((1, PK * H), lambda i: (0, 0)),
              pl.BlockSpec((PK * H, PK * H), lambda i: (0, 0))],
    out_specs=pl.BlockSpec((PBLK, PK * H), lambda i: (i, 0)),
    out_shape=jax.ShapeDtypeStruct((PROWS, PK * H), jnp.float32),
)


def _head_body(p_ref, b2_ref, wd_ref, bd_ref, eps_ref, cmat_ref, smat_ref,
               dmask_ref, ones_ref, o_ref):
    h2 = p_ref[0] + p_ref[1] + b2_ref[...]
    params = jnp.dot(h2, wd_ref[...],
                     preferred_element_type=jnp.float32) + bd_ref[...]
    sp = jax.nn.softplus(params) + 1e-5
    dmask = dmask_ref[...]
    pd = dmask * sp + (1.0 - dmask) * params
    g = jnp.dot(eps_ref[...], cmat_ref[...],
                preferred_element_type=jnp.float32,
                precision=jax.lax.Precision.HIGHEST) + ones_ref[...]
    o_ref[...] = jnp.dot(pd * g, smat_ref[...],
                         preferred_element_type=jnp.float32,
                         precision=jax.lax.Precision.HIGHEST)


_head = pl.pallas_call(
    _head_body,
    grid=(GRID,),
    in_specs=[pl.BlockSpec((2, PBLK, PK * H), lambda i: (0, i, 0)),
              pl.BlockSpec((1, PK * H), lambda i: (0, 0)),
              pl.BlockSpec((PK * H, PK * PARAMS), lambda i: (0, 0)),
              pl.BlockSpec((1, PK * PARAMS), lambda i: (0, 0)),
              pl.BlockSpec((PBLK, PK * LATENT), lambda i: (i, 0)),
              pl.BlockSpec((PK * LATENT, PK * PARAMS), lambda i: (0, 0)),
              pl.BlockSpec((PK * PARAMS, PK * LATENT), lambda i: (0, 0)),
              pl.BlockSpec((1, PK * PARAMS), lambda i: (0, 0)),
              pl.BlockSpec((1, PK * PARAMS), lambda i: (0, 0))],
    out_specs=pl.BlockSpec((PBLK, PK * LATENT), lambda i: (i, 0)),
    out_shape=jax.ShapeDtypeStruct((OROWS, PK * LATENT), jnp.float32),
)

_RI, _CI = np.tril_indices(LATENT)
_CMAT = np.zeros((LATENT, PARAMS), np.float32)
_SMAT = np.zeros((PARAMS, LATENT), np.float32)
_DMASK = np.zeros((1, PARAMS), np.float32)
_ONES = np.zeros((1, PARAMS), np.float32)
_ONES[0, :LATENT] = 1.0
for _k in range(len(_RI)):
    _CMAT[_CI[_k], LATENT + _k] = 1.0
    _SMAT[LATENT + _k, _RI[_k]] = 1.0
    if _RI[_k] == _CI[_k]:
        _DMASK[0, LATENT + _k] = 1.0
for _i in range(LATENT):
    _SMAT[_i, _i] = 1.0

_EYE = np.eye(PK, dtype=np.float32)
_CMAT_P = np.kron(_EYE, _CMAT)
_SMAT_P = np.kron(_EYE, _SMAT)
_DMASK_P = np.tile(_DMASK, (1, PK))
_ONES_P = np.tile(_ONES, (1, PK))


def kernel(x, edge_index, edge_weight, eps, W1, b1, W2, b2, Wd, bd):
    pad = jnp.broadcast_to(jnp.arange(EP - E, dtype=jnp.int32) % N, (2, EP - E))
    ei = jnp.concatenate([edge_index.astype(jnp.int32), pad], axis=1)
    ewp = jnp.concatenate([edge_weight,
                           jnp.zeros((EP - E,), jnp.float32)])
    src = ei[0].reshape(NW, NB, SB)
    dst = ei[1].reshape(NW, NB, SB)
    ew = ewp.reshape(NW, NB, SB)

    eye = jnp.eye(PK, dtype=jnp.float32)
    w2p = jnp.kron(eye, W2)
    wdp = jnp.kron(eye, Wd)
    b1p = jnp.tile(b1, PK).reshape(1, PK * H)
    b2p = jnp.tile(b2, PK).reshape(1, PK * H)
    bdp = jnp.tile(bd, PK).reshape(1, PK * PARAMS)
    epsp = eps.reshape(OROWS, PK * LATENT)

    x3 = x.reshape(OROWS, PK, D)
    w1p = jnp.einsum('df,kfc->kdc', W1, jnp.asarray(_SEL))

    sc_aggregate = _get_sc_aggregate()
    hw1 = _mm_xw1(x3, w1p).reshape(NPAD, H)
    p1 = sc_aggregate(hw1, src, dst, ew).reshape(NC, PROWS, PK * H)
    hw2 = _mid(p1, b1p, w2p).reshape(NPAD, H)
    p2 = sc_aggregate(hw2, src, dst, ew).reshape(NC, PROWS, PK * H)
    out = _head(p2, b2p, wdp, bdp, epsp,
                jnp.asarray(_CMAT_P), jnp.asarray(_SMAT_P),
                jnp.asarray(_DMASK_P), jnp.asarray(_ONES_P))
    return out.reshape(N, LATENT)


# speedup vs baseline: 1.1742x; 1.0007x over previous
import functools

import numpy as np
import jax
import jax.numpy as jnp
from jax import lax
from jax.experimental import pallas as pl
from jax.experimental.pallas import tpu as pltpu
from jax.experimental.pallas import tpu_sc as plsc

N = 10000
E = 320000
D = 128
H = 16
LATENT = 7
PARAMS = 35

NC = 2
NS = 16
NW = NC * NS
SB = 128
EP = 327680
EW = EP // NW
NB = EW // SB
NPAD = 10240
RPT = NPAD // NS
NBUF = 5
NOUT = NB // NBUF

PK = 8
PROWS = NPAD // PK
OROWS = N // PK
PBLK = 128
GRID = PROWS // PBLK


def _sc_body(hw, srcm, dstm, wm, out, acc, src_v, dst_v, w_v, rows_v, msg_v,
             zb_v, gsem, ssem):
    cid = lax.axis_index("c")
    sid = lax.axis_index("s")
    wid = sid * NC + cid

    def _z(i, c):
        zb_v[i, :] = jnp.zeros((H,), jnp.float32)
        return c

    lax.fori_loop(0, RPT, _z, 0)
    pltpu.sync_copy(zb_v, acc.at[pl.ds(sid * RPT, RPT)])
    plsc.subcore_barrier()

    pltpu.sync_copy(srcm.at[wid], src_v)
    pltpu.sync_copy(dstm.at[wid], dst_v)
    pltpu.sync_copy(wm.at[wid], w_v)

    def _mul(j, b):
        for g in range(SB // H):
            w16 = w_v[j, pl.ds(g * H, H)]
            for e in range(H):
                msg_v[b, g * H + e, :] = rows_v[b, g * H + e, :] * w16[e]

    def _gather(j, b):
        pltpu.async_copy(hw.at[src_v.at[j]], rows_v.at[b], gsem.at[b])

    def _gwait(b):
        pltpu.make_async_copy(hw.at[src_v.at[0]], rows_v.at[b],
                              gsem.at[b]).wait()

    def _scatter(j, b):
        pltpu.async_copy(msg_v.at[b], acc.at[dst_v.at[j]], ssem.at[b],
                         add=True)

    def _swait(j, b):
        pltpu.make_async_copy(msg_v.at[b], acc.at[dst_v.at[j]],
                              ssem.at[b]).wait()

    for b in range(NBUF):
        _gather(b, b)
    for b in range(NBUF):
        _gwait(b)
        _mul(b, b)
        _scatter(b, b)
        _gather(b + NBUF, b)

    def _outer(o, c):
        for b in range(NBUF):
            j = o * NBUF + b
            _gwait(b)
            _swait(j, b)
            _mul(j, b)
            _scatter(j, b)
            _gather(j + NBUF, b)
        return c

    lax.fori_loop(1, NOUT - 1, _outer, 0)

    for b in range(NBUF):
        j = (NOUT - 1) * NBUF + b
        _gwait(b)
        _swait(j, b)
        _mul(j, b)
        _scatter(j, b)
    for b in range(NBUF):
        _swait(0, b)
    plsc.subcore_barrier()

    pltpu.sync_copy(acc.at[pl.ds(sid * RPT, RPT)],
                    out.at[cid, pl.ds(sid * RPT, RPT)])


@functools.lru_cache(maxsize=None)
def _get_sc_aggregate():
  return pl.kernel(
    _sc_body,
    out_type=jax.ShapeDtypeStruct((NC, NPAD, H), jnp.float32),
    mesh=plsc.VectorSubcoreMesh(core_axis_name="c", subcore_axis_name="s",
                                num_cores=NC, num_subcores=NS),
    scratch_types=[
        pltpu.VMEM_SHARED((NPAD, H), jnp.float32),
        pltpu.VMEM((NB, SB), jnp.int32),
        pltpu.VMEM((NB, SB), jnp.int32),
        pltpu.VMEM((NB, SB), jnp.float32),
        pltpu.VMEM((NBUF, SB, H), jnp.float32),
        pltpu.VMEM((NBUF, SB, H), jnp.float32),
        pltpu.VMEM((RPT, H), jnp.float32),
        pltpu.SemaphoreType.DMA((NBUF,)),
        pltpu.SemaphoreType.DMA((NBUF,)),
    ],
    compiler_params=pltpu.CompilerParams(use_tc_tiling_on_sc=False),
  )


def _mm_body(xa_ref, xb_ref, w_ref, o_ref):
    def _half(x_ref):
        acc = jnp.dot(x_ref[:, 0, :], w_ref[0],
                      preferred_element_type=jnp.float32)
        for k in range(1, PK):
            acc += jnp.dot(x_ref[:, k, :], w_ref[k],
                           preferred_element_type=jnp.float32)
        return acc

    o_ref[:PBLK // 2, :] = _half(xa_ref)
    o_ref[PBLK // 2:, :] = _half(xb_ref)


_mm_xw1 = pl.pallas_call(
    _mm_body,
    grid=(GRID,),
    in_specs=[pl.BlockSpec((PBLK // 2, PK, D), lambda i: (2 * i, 0, 0)),
              pl.BlockSpec((PBLK // 2, PK, D), lambda i: (2 * i + 1, 0, 0)),
              pl.BlockSpec((PK, D, PK * H), lambda i: (0, 0, 0))],
    out_specs=pl.BlockSpec((PBLK, PK * H), lambda i: (i, 0)),
    out_shape=jax.ShapeDtypeStruct((PROWS, PK * H), jnp.float32),
)

_SEL = np.zeros((PK, H, PK * H), np.float32)
for _k in range(PK):
    for _f in range(H):
        _SEL[_k, _f, _k * H + _f] = 1.0


def _mid_body(p_ref, b1_ref, w2_ref, o_ref):
    h = p_ref[0] + p_ref[1] + b1_ref[...]
    h = jnp.maximum(h, 0.0)
    o_ref[...] = jnp.dot(h, w2_ref[...], preferred_element_type=jnp.float32)


_mid = pl.pallas_call(
    _mid_body,
    grid=(GRID,),
    in_specs=[pl.BlockSpec((2, PBLK, PK * H), lambda i: (0, i, 0)),
              pl.BlockSpec((1, PK * H), lambda i: (0, 0)),
              pl.BlockSpec((PK * H, PK * H), lambda i: (0, 0))],
    out_specs=pl.BlockSpec((PBLK, PK * H), lambda i: (i, 0)),
    out_shape=jax.ShapeDtypeStruct((PROWS, PK * H), jnp.float32),
)


def _head_body(p_ref, b2_ref, wd_ref, bd_ref, eps_ref, cmat_ref, smat_ref,
               dmask_ref, ones_ref, o_ref):
    h2 = p_ref[0] + p_ref[1] + b2_ref[...]
    params = jnp.dot(h2, wd_ref[...],
                     preferred_element_type=jnp.float32) + bd_ref[...]
    sp = jax.nn.softplus(params) + 1e-5
    dmask = dmask_ref[...]
    pd = dmask * sp + (1.0 - dmask) * params
    g = jnp.dot(eps_ref[...], cmat_ref[...],
                preferred_element_type=jnp.float32,
                precision=jax.lax.Precision.HIGHEST) + ones_ref[...]
    o_ref[...] = jnp.dot(pd * g, smat_ref[...],
                         preferred_element_type=jnp.float32,
                         precision=jax.lax.Precision.HIGHEST)


_head = pl.pallas_call(
    _head_body,
    grid=(GRID,),
    in_specs=[pl.BlockSpec((2, PBLK, PK * H), lambda i: (0, i, 0)),
              pl.BlockSpec((1, PK * H), lambda i: (0, 0)),
              pl.BlockSpec((PK * H, PK * PARAMS), lambda i: (0, 0)),
              pl.BlockSpec((1, PK * PARAMS), lambda i: (0, 0)),
              pl.BlockSpec((PBLK, PK * LATENT), lambda i: (i, 0)),
              pl.BlockSpec((PK * LATENT, PK * PARAMS), lambda i: (0, 0)),
              pl.BlockSpec((PK * PARAMS, PK * LATENT), lambda i: (0, 0)),
              pl.BlockSpec((1, PK * PARAMS), lambda i: (0, 0)),
              pl.BlockSpec((1, PK * PARAMS), lambda i: (0, 0))],
    out_specs=pl.BlockSpec((PBLK, PK * LATENT), lambda i: (i, 0)),
    out_shape=jax.ShapeDtypeStruct((OROWS, PK * LATENT), jnp.float32),
)

_RI, _CI = np.tril_indices(LATENT)
_CMAT = np.zeros((LATENT, PARAMS), np.float32)
_SMAT = np.zeros((PARAMS, LATENT), np.float32)
_DMASK = np.zeros((1, PARAMS), np.float32)
_ONES = np.zeros((1, PARAMS), np.float32)
_ONES[0, :LATENT] = 1.0
for _k in range(len(_RI)):
    _CMAT[_CI[_k], LATENT + _k] = 1.0
    _SMAT[LATENT + _k, _RI[_k]] = 1.0
    if _RI[_k] == _CI[_k]:
        _DMASK[0, LATENT + _k] = 1.0
for _i in range(LATENT):
    _SMAT[_i, _i] = 1.0

_EYE = np.eye(PK, dtype=np.float32)
_CMAT_P = np.kron(_EYE, _CMAT)
_SMAT_P = np.kron(_EYE, _SMAT)
_DMASK_P = np.tile(_DMASK, (1, PK))
_ONES_P = np.tile(_ONES, (1, PK))


def kernel(x, edge_index, edge_weight, eps, W1, b1, W2, b2, Wd, bd):
    pad = jnp.broadcast_to(jnp.arange(EP - E, dtype=jnp.int32) % N, (2, EP - E))
    ei = jnp.concatenate([edge_index.astype(jnp.int32), pad], axis=1)
    ewp = jnp.concatenate([edge_weight,
                           jnp.zeros((EP - E,), jnp.float32)])
    src = ei[0].reshape(NW, NB, SB)
    dst = ei[1].reshape(NW, NB, SB)
    ew = ewp.reshape(NW, NB, SB)

    eye = jnp.eye(PK, dtype=jnp.float32)
    w2p = jnp.kron(eye, W2)
    wdp = jnp.kron(eye, Wd)
    b1p = jnp.tile(b1, PK).reshape(1, PK * H)
    b2p = jnp.tile(b2, PK).reshape(1, PK * H)
    bdp = jnp.tile(bd, PK).reshape(1, PK * PARAMS)
    epsp = eps.reshape(OROWS, PK * LATENT)

    x3 = x.reshape(OROWS, PK, D)
    w1p = jnp.einsum('df,kfc->kdc', W1, jnp.asarray(_SEL))

    sc_aggregate = _get_sc_aggregate()
    hw1 = _mm_xw1(x3, x3, w1p).reshape(NPAD, H)
    p1 = sc_aggregate(hw1, src, dst, ew).reshape(NC, PROWS, PK * H)
    hw2 = _mid(p1, b1p, w2p).reshape(NPAD, H)
    p2 = sc_aggregate(hw2, src, dst, ew).reshape(NC, PROWS, PK * H)
    out = _head(p2, b2p, wdp, bdp, epsp,
                jnp.asarray(_CMAT_P), jnp.asarray(_SMAT_P),
                jnp.asarray(_DMASK_P), jnp.asarray(_ONES_P))
    return out.reshape(N, LATENT)


# speedup vs baseline: 1.1758x; 1.0014x over previous
import functools

import numpy as np
import jax
import jax.numpy as jnp
from jax import lax
from jax.experimental import pallas as pl
from jax.experimental.pallas import tpu as pltpu
from jax.experimental.pallas import tpu_sc as plsc

N = 10000
E = 320000
D = 128
H = 16
LATENT = 7
PARAMS = 35

NC = 2
NS = 16
NW = NC * NS
SB = 128
EP = 327680
EW = EP // NW
NB = EW // SB
NPAD = 10240
RPT = NPAD // NS
NBUF = 5
NOUT = NB // NBUF

PK = 8
PROWS = NPAD // PK
OROWS = N // PK
PBLK = 128
GRID = PROWS // PBLK


def _sc_body(hw, srcm, dstm, wm, out, acc, src_v, dst_v, w_v, rows_v, msg_v,
             zb_v, gsem, ssem):
    cid = lax.axis_index("c")
    sid = lax.axis_index("s")
    wid = sid * NC + cid

    def _z(i, c):
        zb_v[i, :] = jnp.zeros((H,), jnp.float32)
        return c

    lax.fori_loop(0, RPT, _z, 0)
    pltpu.sync_copy(zb_v, acc.at[pl.ds(sid * RPT, RPT)])
    plsc.subcore_barrier()

    pltpu.sync_copy(srcm.at[wid], src_v)
    pltpu.sync_copy(dstm.at[wid], dst_v)
    pltpu.sync_copy(wm.at[wid], w_v)

    def _mul(j, b):
        for g in range(SB // H):
            w16 = w_v[j, pl.ds(g * H, H)]
            for e in range(H):
                msg_v[b, g * H + e, :] = rows_v[b, g * H + e, :] * w16[e]

    def _gather(j, b):
        pltpu.async_copy(hw.at[src_v.at[j]], rows_v.at[b], gsem.at[b])

    def _gwait(b):
        pltpu.make_async_copy(hw.at[src_v.at[0]], rows_v.at[b],
                              gsem.at[b]).wait()

    def _scatter(j, b):
        pltpu.async_copy(msg_v.at[b], acc.at[dst_v.at[j]], ssem.at[b],
                         add=True)

    def _swait(j, b):
        pltpu.make_async_copy(msg_v.at[b], acc.at[dst_v.at[j]],
                              ssem.at[b]).wait()

    for b in range(NBUF):
        _gather(b, b)
    for b in range(NBUF):
        _gwait(b)
        _mul(b, b)
        _scatter(b, b)
        _gather(b + NBUF, b)

    def _outer(o, c):
        for b in range(NBUF):
            j = o * NBUF + b
            _gwait(b)
            _swait(j, b)
            _mul(j, b)
            _scatter(j, b)
            _gather(j + NBUF, b)
        return c

    lax.fori_loop(1, NOUT - 1, _outer, 0)

    for b in range(NBUF):
        j = (NOUT - 1) * NBUF + b
        _gwait(b)
        _swait(j, b)
        _mul(j, b)
        _scatter(j, b)
    for b in range(NBUF):
        _swait(0, b)
    plsc.subcore_barrier()

    pltpu.sync_copy(acc.at[pl.ds(sid * RPT, RPT)],
                    out.at[cid, pl.ds(sid * RPT, RPT)])


@functools.lru_cache(maxsize=None)
def _get_sc_aggregate():
  return pl.kernel(
    _sc_body,
    out_type=jax.ShapeDtypeStruct((NC, NPAD, H), jnp.float32),
    mesh=plsc.VectorSubcoreMesh(core_axis_name="c", subcore_axis_name="s",
                                num_cores=NC, num_subcores=NS),
    scratch_types=[
        pltpu.VMEM_SHARED((NPAD, H), jnp.float32),
        pltpu.VMEM((NB, SB), jnp.int32),
        pltpu.VMEM((NB, SB), jnp.int32),
        pltpu.VMEM((NB, SB), jnp.float32),
        pltpu.VMEM((NBUF, SB, H), jnp.float32),
        pltpu.VMEM((NBUF, SB, H), jnp.float32),
        pltpu.VMEM((RPT, H), jnp.float32),
        pltpu.SemaphoreType.DMA((NBUF,)),
        pltpu.SemaphoreType.DMA((NBUF,)),
    ],
    compiler_params=pltpu.CompilerParams(use_tc_tiling_on_sc=False),
  )


def _mm_body(x_ref, w_ref, o_ref):
    acc = jnp.dot(x_ref[:, 0, :], w_ref[0],
                  preferred_element_type=jnp.float32)
    for k in range(1, PK):
        acc += jnp.dot(x_ref[:, k, :], w_ref[k],
                       preferred_element_type=jnp.float32)
    o_ref[...] = acc


_mm_xw1 = pl.pallas_call(
    _mm_body,
    grid=(GRID,),
    in_specs=[pl.BlockSpec((PBLK, PK, D), lambda i: (i, 0, 0)),
              pl.BlockSpec((PK, D, PK * H), lambda i: (0, 0, 0))],
    out_specs=pl.BlockSpec((PBLK, PK * H), lambda i: (i, 0)),
    out_shape=jax.ShapeDtypeStruct((PROWS, PK * H), jnp.float32),
)

_SEL = np.zeros((PK, H, PK * H), np.float32)
for _k in range(PK):
    for _f in range(H):
        _SEL[_k, _f, _k * H + _f] = 1.0


def _mid_body(p_ref, b1_ref, w2_ref, o_ref):
    h = p_ref[0] + p_ref[1] + b1_ref[...]
    h = jnp.maximum(h, 0.0)
    o_ref[...] = jnp.dot(h, w2_ref[...], preferred_element_type=jnp.float32)


_mid = pl.pallas_call(
    _mid_body,
    grid=(GRID,),
    in_specs=[pl.BlockSpec((2, PBLK, PK * H), lambda i: (0, i, 0)),
              pl.BlockSpec((1, PK * H), lambda i: (0, 0)),
              pl.BlockSpec((PK * H, PK * H), lambda i: (0, 0))],
    out_specs=pl.BlockSpec((PBLK, PK * H), lambda i: (i, 0)),
    out_shape=jax.ShapeDtypeStruct((PROWS, PK * H), jnp.float32),
)


def _head_body(p_ref, b2_ref, wd_ref, bd_ref, eps_ref, cmat_ref, smat_ref,
               dmask_ref, ones_ref, o_ref):
    h2 = p_ref[0] + p_ref[1] + b2_ref[...]
    params = jnp.dot(h2, wd_ref[...],
                     preferred_element_type=jnp.float32) + bd_ref[...]
    sp = jax.nn.softplus(params) + 1e-5
    dmask = dmask_ref[...]
    pd = dmask * sp + (1.0 - dmask) * params
    g = jnp.dot(eps_ref[...], cmat_ref[...],
                preferred_element_type=jnp.float32,
                precision=jax.lax.Precision.HIGHEST) + ones_ref[...]
    o_ref[...] = jnp.dot(pd * g, smat_ref[...],
                         preferred_element_type=jnp.float32,
                         precision=jax.lax.Precision.HIGHEST)


_head = pl.pallas_call(
    _head_body,
    grid=(GRID,),
    in_specs=[pl.BlockSpec((2, PBLK, PK * H), lambda i: (0, i, 0)),
              pl.BlockSpec((1, PK * H), lambda i: (0, 0)),
              pl.BlockSpec((PK * H, PK * PARAMS), lambda i: (0, 0)),
              pl.BlockSpec((1, PK * PARAMS), lambda i: (0, 0)),
              pl.BlockSpec((PBLK, PK * LATENT), lambda i: (i, 0)),
              pl.BlockSpec((PK * LATENT, PK * PARAMS), lambda i: (0, 0)),
              pl.BlockSpec((PK * PARAMS, PK * LATENT), lambda i: (0, 0)),
              pl.BlockSpec((1, PK * PARAMS), lambda i: (0, 0)),
              pl.BlockSpec((1, PK * PARAMS), lambda i: (0, 0))],
    out_specs=pl.BlockSpec((PBLK, PK * LATENT), lambda i: (i, 0)),
    out_shape=jax.ShapeDtypeStruct((OROWS, PK * LATENT), jnp.float32),
)

_RI, _CI = np.tril_indices(LATENT)
_CMAT = np.zeros((LATENT, PARAMS), np.float32)
_SMAT = np.zeros((PARAMS, LATENT), np.float32)
_DMASK = np.zeros((1, PARAMS), np.float32)
_ONES = np.zeros((1, PARAMS), np.float32)
_ONES[0, :LATENT] = 1.0
for _k in range(len(_RI)):
    _CMAT[_CI[_k], LATENT + _k] = 1.0
    _SMAT[LATENT + _k, _RI[_k]] = 1.0
    if _RI[_k] == _CI[_k]:
        _DMASK[0, LATENT + _k] = 1.0
for _i in range(LATENT):
    _SMAT[_i, _i] = 1.0

_EYE = np.eye(PK, dtype=np.float32)
_CMAT_P = np.kron(_EYE, _CMAT)
_SMAT_P = np.kron(_EYE, _SMAT)
_DMASK_P = np.tile(_DMASK, (1, PK))
_ONES_P = np.tile(_ONES, (1, PK))


def kernel(x, edge_index, edge_weight, eps, W1, b1, W2, b2, Wd, bd):
    pad = jnp.broadcast_to(jnp.arange(EP - E, dtype=jnp.int32) % N, (2, EP - E))
    ei = jnp.concatenate([edge_index.astype(jnp.int32), pad], axis=1)
    ewp = jnp.concatenate([edge_weight,
                           jnp.zeros((EP - E,), jnp.float32)])
    src = ei[0].reshape(NW, NB, SB)
    dst = ei[1].reshape(NW, NB, SB)
    ew = ewp.reshape(NW, NB, SB)

    eye = jnp.eye(PK, dtype=jnp.float32)
    w2p = jnp.kron(eye, W2)
    wdp = jnp.kron(eye, Wd)
    b1p = jnp.tile(b1, PK).reshape(1, PK * H)
    b2p = jnp.tile(b2, PK).reshape(1, PK * H)
    bdp = jnp.tile(bd, PK).reshape(1, PK * PARAMS)
    epsp = eps.reshape(OROWS, PK * LATENT)

    x3 = x.reshape(OROWS, PK, D)
    w1p = jnp.einsum('df,kfc->kdc', W1, jnp.asarray(_SEL))

    sc_aggregate = _get_sc_aggregate()
    hw1 = _mm_xw1(x3, w1p).reshape(NPAD, H)
    p1 = sc_aggregate(hw1, src, dst, ew).reshape(NC, PROWS, PK * H)
    hw2 = _mid(p1, b1p, w2p).reshape(NPAD, H)
    p2 = sc_aggregate(hw2, src, dst, ew).reshape(NC, PROWS, PK * H)
    out = _head(p2, b2p, wdp, bdp, epsp,
                jnp.asarray(_CMAT_P), jnp.asarray(_SMAT_P),
                jnp.asarray(_DMASK_P), jnp.asarray(_ONES_P))
    return out.reshape(N, LATENT)
